# Initial kernel scaffold; baseline (speedup 1.0000x reference)
import jax
import jax.numpy as jnp
from jax.experimental import pallas as pl


def kernel(feat, grid_index):
    raise NotImplementedError("write your pallas kernel here")


# speedup vs baseline: 1.7002x; 1.7002x over previous
import jax
import jax.numpy as jnp
from jax import lax
from jax.experimental import pallas as pl
from jax.experimental.pallas import tpu as pltpu
from jax.experimental.pallas import tpu_sc as plsc

N_PTS = 524288
C_DIM = 96
N_CELLS = 262144
NG = 8
G_SHIFT = 15
GB = 8
NB = NG * GB
BUCKET = N_CELLS // NB
NC = 2
NS = 16
P = N_PTS // NS
GCAP = P + 16
MCAP = P + 128
CHUNK = 64
TS = BUCKET // NS
DC = 128
SB = 2048


def _body(feat_hbm, gi_hbm, out_hbm,
          gbuf, glist, mptr, lcnt, idbuf, gidx, lidx, rowbuf, dbuf, zbuf,
          zc, ctmp2, inv, acc, scnt, sem):
    c = lax.axis_index("c")
    s = lax.axis_index("s")
    tbase = s * P
    iota16 = lax.iota(jnp.int32, 16)
    zeros16 = jnp.zeros((16,), jnp.float32)
    ones16 = jnp.ones((16,), jnp.float32)
    neg16 = jnp.full((16,), -1, jnp.int32)

    def zrow(r, _):
        for q in range(C_DIM // 16):
            zbuf[r, pl.ds(q * 16, 16)] = zeros16
        return 0
    lax.fori_loop(0, DC, zrow, 0)

    def zcrow(r, _):
        zc[r] = zeros16
        return 0
    lax.fori_loop(0, 16, zcrow, 0)

    def idrow(q, _):
        for i in range(8):
            idbuf[q, pl.ds(i * 16, 16)] = q * 128 + i * 16 + iota16
        return 0
    lax.fori_loop(0, (BUCKET // 16) // 128, idrow, 0)

    def group_body(gi, _):
        g = c * (NG // NC) + gi

        def stream_body(ch, gcnt):
            pltpu.sync_copy(gi_hbm.at[pl.ds(tbase + ch * SB, SB)], gbuf)

            def scan_body(i, cnt):
                v = gbuf[pl.ds(i * 16, 16)]
                m = jnp.right_shift(v, G_SHIFT) == g
                rel = ch * SB + i * 16 + iota16
                e = jnp.bitwise_or(
                    jnp.left_shift(jnp.bitwise_and(v, 32767), 15), rel)
                pos = cnt + plsc.cumsum(m.astype(jnp.int32)) - 1
                plsc.store_scatter(glist, [pos], e, mask=m)
                return cnt + jnp.sum(m.astype(jnp.int32))
            return lax.fori_loop(0, SB // 16, scan_body, gcnt)
        gcnt = lax.fori_loop(0, P // SB, stream_body, jnp.int32(0))
        plsc.store_scatter(glist, [gcnt + iota16], neg16)
        gsteps = jnp.right_shift(gcnt + 15, 4)

        def bucket_body(sub, _):
            b = g * GB + sub

            def zl(i, _):
                lcnt[i] = zeros16
                return 0
            lax.fori_loop(0, BUCKET // 16, zl, 0)
            for kk in range(TS // DC):
                pltpu.sync_copy(zbuf, acc.at[pl.ds(s * TS + kk * DC, DC)])
            pltpu.sync_copy(zc, scnt.at[pl.ds(s * 16, 16)])
            plsc.subcore_barrier()

            def bscan(i, cnt):
                e = glist[pl.ds(i * 16, 16)]
                cell15 = jnp.right_shift(e, 15)
                m = jnp.logical_and(
                    jnp.right_shift(cell15, 12) == sub, e >= 0)
                loc = jnp.bitwise_and(cell15, BUCKET - 1)
                plsc.addupdate_scatter(
                    lcnt, [jnp.right_shift(loc, 4),
                           jnp.bitwise_and(loc, 15)], ones16, mask=m)
                packed = jnp.bitwise_or(
                    jnp.left_shift(loc, 15), jnp.bitwise_and(e, 32767))
                pos = cnt + plsc.cumsum(m.astype(jnp.int32)) - 1
                plsc.store_scatter(mptr, [pos], packed, mask=m)
                return cnt + jnp.sum(m.astype(jnp.int32))
            cnt = lax.fori_loop(0, gsteps, bscan, jnp.int32(0))

            def pad_body(q, _):
                plsc.store_scatter(mptr, [cnt + q * 16 + iota16], neg16)
                return 0
            lax.fori_loop(0, 4, pad_body, 0)

            ncnk = jnp.right_shift(cnt + (CHUNK - 1), 6)

            def flush_body(j, _):
                for q in range(CHUNK // 16):
                    r16 = mptr[pl.ds(j * CHUNK + q * 16, 16)]
                    valid = r16 >= 0
                    loc = jnp.where(valid,
                                    jnp.right_shift(r16, 15),
                                    jnp.int32(BUCKET))
                    rel = jnp.bitwise_and(r16, 32767)
                    gidx[pl.ds(q * 16, 16)] = tbase + rel
                    lidx[pl.ds(q * 16, 16)] = loc
                pltpu.async_copy(feat_hbm.at[gidx], rowbuf, sem).wait()
                pltpu.sync_copy(rowbuf, acc.at[lidx], add=True)
                return 0
            lax.fori_loop(0, ncnk, flush_body, 0)

            for q in range((BUCKET // 16) // 128):
                pltpu.sync_copy(lcnt.at[pl.ds(q * 128, 128)],
                                scnt.at[idbuf.at[q]], add=True)
            plsc.subcore_barrier()

            pltpu.sync_copy(scnt.at[pl.ds(s * 16, 16)], ctmp2)

            def invb(i, _):
                cv = ctmp2[i]
                inv[pl.ds(i * 16, 16)] = 1.0 / jnp.maximum(cv, 1.0)
                return 0
            lax.fori_loop(0, 16, invb, 0)

            def dchunk(kk, _):
                row0 = s * TS + kk * DC
                pltpu.sync_copy(acc.at[pl.ds(row0, DC)], dbuf)

                def drow(r, _):
                    ivv = plsc.load_gather(
                        inv, [jnp.full((16,), kk * DC + r, jnp.int32)])
                    for q in range(C_DIM // 16):
                        dbuf[r, pl.ds(q * 16, 16)] = (
                            dbuf[r, pl.ds(q * 16, 16)] * ivv)
                    return 0
                lax.fori_loop(0, DC, drow, 0)
                pltpu.sync_copy(dbuf,
                                out_hbm.at[pl.ds(b * BUCKET + row0, DC)])
                return 0
            lax.fori_loop(0, TS // DC, dchunk, 0)

            plsc.subcore_barrier()
            return 0
        lax.fori_loop(0, GB, bucket_body, 0)
        return 0
    lax.fori_loop(0, NG // NC, group_body, 0)


@jax.jit
def kernel(feat, grid_index):
    run = pl.kernel(
        _body,
        out_type=jax.ShapeDtypeStruct((N_CELLS, C_DIM), jnp.float32),
        mesh=plsc.VectorSubcoreMesh(core_axis_name="c", subcore_axis_name="s"),
        compiler_params=pltpu.CompilerParams(
            needs_layout_passes=False, use_tc_tiling_on_sc=False),
        scratch_types=[
            pltpu.VMEM((SB,), jnp.int32),
            pltpu.VMEM((GCAP,), jnp.int32),
            pltpu.VMEM((MCAP,), jnp.int32),
            pltpu.VMEM((BUCKET // 16, 16), jnp.float32),
            pltpu.VMEM(((BUCKET // 16) // 128, 128), jnp.int32),
            pltpu.VMEM((CHUNK,), jnp.int32),
            pltpu.VMEM((CHUNK,), jnp.int32),
            pltpu.VMEM((CHUNK, C_DIM), jnp.float32),
            pltpu.VMEM((DC, C_DIM), jnp.float32),
            pltpu.VMEM((DC, C_DIM), jnp.float32),
            pltpu.VMEM((16, 16), jnp.float32),
            pltpu.VMEM((16, 16), jnp.float32),
            pltpu.VMEM((TS,), jnp.float32),
            pltpu.VMEM_SHARED((BUCKET + 8, C_DIM), jnp.float32),
            pltpu.VMEM_SHARED((BUCKET // 16, 16), jnp.float32),
            pltpu.SemaphoreType.DMA,
        ],
    )
    return run(feat, grid_index)
